# baseline (speedup 1.0000x reference)
import jax
import jax.numpy as jnp
from jax.experimental import pallas as pl
from jax.experimental.pallas import tpu as pltpu

N_EXPERT = 8
TOP_K = 2
ROWS = 256


def _router_body(x_ref, gw_ref, o_ref, xbf_ref):
    x = x_ref[...]
    o_ref[...] = jax.lax.dot_general(
        x, gw_ref[...], (((1,), (1,)), ((), ())),
        preferred_element_type=jnp.float32)
    xbf_ref[...] = x.astype(jnp.bfloat16)


def _fc12_body(be_ref, x_ref, w1_ref, w2_ref, h_ref):
    x = x_ref[...].astype(jnp.float32)
    h1 = jax.lax.dot_general(x, w1_ref[0], (((1,), (1,)), ((), ())),
                             preferred_element_type=jnp.float32)
    h2 = jax.lax.dot_general(x, w2_ref[0], (((1,), (1,)), ((), ())),
                             preferred_element_type=jnp.float32)
    h_ref[...] = (jax.nn.silu(h1) * h2).astype(jnp.bfloat16)


def _proj_body(be_ref, h_ref, wp_ref, wt_ref, o_ref):
    h = h_ref[...].astype(jnp.float32)
    out = jax.lax.dot_general(h, wp_ref[0], (((1,), (1,)), ((), ())),
                              preferred_element_type=jnp.float32)
    o_ref[...] = (out * wt_ref[0, 0].reshape(-1, 1)).astype(jnp.bfloat16)


def _shared_body(x_ref, w1_ref, w2_ref, wp_ref, r_ref, o_ref):
    x = x_ref[...]
    h1 = jax.lax.dot_general(x, w1_ref[...], (((1,), (1,)), ((), ())),
                             preferred_element_type=jnp.float32)
    h2 = jax.lax.dot_general(x, w2_ref[...], (((1,), (1,)), ((), ())),
                             preferred_element_type=jnp.float32)
    h = (jax.nn.silu(h1) * h2).astype(jnp.bfloat16)
    out = jax.lax.dot_general(h, wp_ref[...], (((1,), (1,)), ((), ())),
                              preferred_element_type=jnp.float32)
    routed = r_ref[...].astype(jnp.float32)
    o_ref[...] = out + routed[:, 0, :] + routed[:, 1, :]


def kernel(x, gate_w, fc1_w, fc2_w, proj_w, s_fc1_w, s_fc2_w, s_proj_w):
    B, T, D = x.shape
    N = B * T
    F = fc1_w.shape[1]
    A = N * TOP_K
    PAD = A + N_EXPERT * ROWS
    n_blocks = PAD // ROWS

    x_flat = x.reshape(N, D)

    logits, x_bf = pl.pallas_call(
        _router_body,
        grid=(N // 512,),
        in_specs=[pl.BlockSpec((512, D), lambda i: (i, 0)),
                  pl.BlockSpec((N_EXPERT, D), lambda i: (0, 0))],
        out_specs=[pl.BlockSpec((512, N_EXPERT), lambda i: (i, 0)),
                   pl.BlockSpec((512, D), lambda i: (i, 0))],
        out_shape=[jax.ShapeDtypeStruct((N, N_EXPERT), jnp.float32),
                   jax.ShapeDtypeStruct((N, D), jnp.bfloat16)],
    )(x_flat, gate_w)

    top_vals, top_idx = jax.lax.top_k(logits, TOP_K)
    probs = jax.nn.softmax(top_vals, axis=-1)

    ew = top_idx.reshape(-1).astype(jnp.int32)
    wts = probs.reshape(-1)
    tok = jax.lax.iota(jnp.int32, A) // TOP_K

    order = jnp.argsort(ew, stable=True)
    ew_s = ew[order]
    counts = jnp.bincount(ew, length=N_EXPERT)
    padded = ((counts + ROWS - 1) // ROWS) * ROWS
    pstart = jnp.concatenate([jnp.zeros((1,), jnp.int32),
                              jnp.cumsum(padded).astype(jnp.int32)])
    start = jnp.concatenate([jnp.zeros((1,), jnp.int32),
                             jnp.cumsum(counts).astype(jnp.int32)])
    rank = jax.lax.iota(jnp.int32, A) - start[ew_s]
    dest = pstart[ew_s] + rank

    tok_pad = jnp.zeros((PAD,), jnp.int32).at[dest].set(tok[order])
    wt_pad = jnp.zeros((PAD,), jnp.float32).at[dest].set(wts[order])
    inv = jnp.zeros((A,), jnp.int32).at[order].set(dest)

    blk_ids = jax.lax.iota(jnp.int32, n_blocks) * ROWS
    block_expert = jnp.minimum(
        jnp.searchsorted(pstart[1:], blk_ids, side="right"),
        N_EXPERT - 1).astype(jnp.int32)

    x_g = x_bf[tok_pad]

    h_pad = pl.pallas_call(
        _fc12_body,
        grid_spec=pltpu.PrefetchScalarGridSpec(
            num_scalar_prefetch=1,
            grid=(n_blocks,),
            in_specs=[
                pl.BlockSpec((ROWS, D), lambda i, be: (i, 0)),
                pl.BlockSpec((1, F, D), lambda i, be: (be[i], 0, 0)),
                pl.BlockSpec((1, F, D), lambda i, be: (be[i], 0, 0)),
            ],
            out_specs=pl.BlockSpec((ROWS, F), lambda i, be: (i, 0)),
        ),
        out_shape=jax.ShapeDtypeStruct((PAD, F), jnp.bfloat16),
        compiler_params=pltpu.CompilerParams(
            dimension_semantics=("arbitrary",)),
    )(block_expert, x_g, fc1_w, fc2_w)

    out_pad = pl.pallas_call(
        _proj_body,
        grid_spec=pltpu.PrefetchScalarGridSpec(
            num_scalar_prefetch=1,
            grid=(n_blocks,),
            in_specs=[
                pl.BlockSpec((ROWS, F), lambda i, be: (i, 0)),
                pl.BlockSpec((1, D, F), lambda i, be: (be[i], 0, 0)),
                pl.BlockSpec((1, 1, ROWS), lambda i, be: (i, 0, 0)),
            ],
            out_specs=pl.BlockSpec((ROWS, D), lambda i, be: (i, 0)),
        ),
        out_shape=jax.ShapeDtypeStruct((PAD, D), jnp.bfloat16),
        compiler_params=pltpu.CompilerParams(
            dimension_semantics=("arbitrary",)),
    )(block_expert, h_pad, proj_w, wt_pad.reshape(n_blocks, 1, ROWS))

    routed_pairs = out_pad[inv].reshape(N, TOP_K, D)

    n_sblocks = N // ROWS
    y = pl.pallas_call(
        _shared_body,
        grid=(n_sblocks,),
        in_specs=[
            pl.BlockSpec((ROWS, D), lambda i: (i, 0)),
            pl.BlockSpec((F, D), lambda i: (0, 0)),
            pl.BlockSpec((F, D), lambda i: (0, 0)),
            pl.BlockSpec((D, F), lambda i: (0, 0)),
            pl.BlockSpec((ROWS, TOP_K, D), lambda i: (i, 0, 0)),
        ],
        out_specs=pl.BlockSpec((ROWS, D), lambda i: (i, 0)),
        out_shape=jax.ShapeDtypeStruct((N, D), jnp.float32),
        compiler_params=pltpu.CompilerParams(
            dimension_semantics=("arbitrary",)),
    )(x_bf, s_fc1_w.astype(jnp.bfloat16),
      s_fc2_w.astype(jnp.bfloat16), s_proj_w.astype(jnp.bfloat16),
      routed_pairs)

    return y.reshape(B, T, D)


# speedup vs baseline: 1.1755x; 1.1755x over previous
import jax
import jax.numpy as jnp
from jax.experimental import pallas as pl
from jax.experimental.pallas import tpu as pltpu

N_EXPERT = 8
TOP_K = 2
ROWS = 256


def _router_body(x_ref, gw_ref, o_ref):
    o_ref[...] = jax.lax.dot_general(
        x_ref[...], gw_ref[...], (((1,), (1,)), ((), ())),
        preferred_element_type=jnp.float32)


def _fc12_body(be_ref, x_ref, w1_ref, w2_ref, h_ref):
    x = x_ref[...]
    h1 = jax.lax.dot_general(x, w1_ref[0], (((1,), (1,)), ((), ())),
                             preferred_element_type=jnp.float32)
    h2 = jax.lax.dot_general(x, w2_ref[0], (((1,), (1,)), ((), ())),
                             preferred_element_type=jnp.float32)
    h_ref[...] = jax.nn.silu(h1) * h2


def _proj_body(be_ref, h_ref, wp_ref, wt_ref, o_ref):
    out = jax.lax.dot_general(h_ref[...], wp_ref[0], (((1,), (1,)), ((), ())),
                              preferred_element_type=jnp.float32)
    o_ref[...] = out * wt_ref[0, 0].reshape(-1, 1)


def _shared_body(x_ref, w1_ref, w2_ref, wp_ref, r_ref, o_ref):
    x = x_ref[...]
    h1 = jax.lax.dot_general(x, w1_ref[...], (((1,), (1,)), ((), ())),
                             preferred_element_type=jnp.float32)
    h2 = jax.lax.dot_general(x, w2_ref[...], (((1,), (1,)), ((), ())),
                             preferred_element_type=jnp.float32)
    h = jax.nn.silu(h1) * h2
    out = jax.lax.dot_general(h, wp_ref[...], (((1,), (1,)), ((), ())),
                              preferred_element_type=jnp.float32)
    routed = r_ref[...]
    o_ref[...] = out + routed[:, 0, :] + routed[:, 1, :]


def kernel(x, gate_w, fc1_w, fc2_w, proj_w, s_fc1_w, s_fc2_w, s_proj_w):
    B, T, D = x.shape
    N = B * T
    F = fc1_w.shape[1]
    A = N * TOP_K
    PAD = A + N_EXPERT * ROWS
    n_blocks = PAD // ROWS

    x_flat = x.reshape(N, D)

    logits = pl.pallas_call(
        _router_body,
        grid=(N // 512,),
        in_specs=[pl.BlockSpec((512, D), lambda i: (i, 0)),
                  pl.BlockSpec((N_EXPERT, D), lambda i: (0, 0))],
        out_specs=pl.BlockSpec((512, N_EXPERT), lambda i: (i, 0)),
        out_shape=jax.ShapeDtypeStruct((N, N_EXPERT), jnp.float32),
    )(x_flat, gate_w)

    top_vals, top_idx = jax.lax.top_k(logits, TOP_K)
    probs = jax.nn.softmax(top_vals, axis=-1)

    ew = top_idx.reshape(-1).astype(jnp.int32)
    wts = probs.reshape(-1)
    tok = jax.lax.iota(jnp.int32, A) // TOP_K

    order = jnp.argsort(ew, stable=True)
    ew_s = ew[order]
    counts = jnp.bincount(ew, length=N_EXPERT)
    padded = ((counts + ROWS - 1) // ROWS) * ROWS
    pstart = jnp.concatenate([jnp.zeros((1,), jnp.int32),
                              jnp.cumsum(padded).astype(jnp.int32)])
    start = jnp.concatenate([jnp.zeros((1,), jnp.int32),
                             jnp.cumsum(counts).astype(jnp.int32)])
    rank = jax.lax.iota(jnp.int32, A) - start[ew_s]
    dest = pstart[ew_s] + rank

    tok_pad = jnp.zeros((PAD,), jnp.int32).at[dest].set(tok[order])
    wt_pad = jnp.zeros((PAD,), jnp.float32).at[dest].set(wts[order])
    inv = jnp.zeros((A,), jnp.int32).at[order].set(dest)

    blk_ids = jax.lax.iota(jnp.int32, n_blocks) * ROWS
    block_expert = jnp.minimum(
        jnp.searchsorted(pstart[1:], blk_ids, side="right"),
        N_EXPERT - 1).astype(jnp.int32)

    x_g = x_flat[tok_pad]

    h_pad = pl.pallas_call(
        _fc12_body,
        grid_spec=pltpu.PrefetchScalarGridSpec(
            num_scalar_prefetch=1,
            grid=(n_blocks,),
            in_specs=[
                pl.BlockSpec((ROWS, D), lambda i, be: (i, 0)),
                pl.BlockSpec((1, F, D), lambda i, be: (be[i], 0, 0)),
                pl.BlockSpec((1, F, D), lambda i, be: (be[i], 0, 0)),
            ],
            out_specs=pl.BlockSpec((ROWS, F), lambda i, be: (i, 0)),
        ),
        out_shape=jax.ShapeDtypeStruct((PAD, F), jnp.float32),
        compiler_params=pltpu.CompilerParams(
            dimension_semantics=("arbitrary",)),
    )(block_expert, x_g, fc1_w, fc2_w)

    out_pad = pl.pallas_call(
        _proj_body,
        grid_spec=pltpu.PrefetchScalarGridSpec(
            num_scalar_prefetch=1,
            grid=(n_blocks,),
            in_specs=[
                pl.BlockSpec((ROWS, F), lambda i, be: (i, 0)),
                pl.BlockSpec((1, D, F), lambda i, be: (be[i], 0, 0)),
                pl.BlockSpec((1, 1, ROWS), lambda i, be: (i, 0, 0)),
            ],
            out_specs=pl.BlockSpec((ROWS, D), lambda i, be: (i, 0)),
        ),
        out_shape=jax.ShapeDtypeStruct((PAD, D), jnp.float32),
        compiler_params=pltpu.CompilerParams(
            dimension_semantics=("arbitrary",)),
    )(block_expert, h_pad, proj_w, wt_pad.reshape(n_blocks, 1, ROWS))

    routed_pairs = out_pad[inv].reshape(N, TOP_K, D)

    n_sblocks = N // ROWS
    y = pl.pallas_call(
        _shared_body,
        grid=(n_sblocks,),
        in_specs=[
            pl.BlockSpec((ROWS, D), lambda i: (i, 0)),
            pl.BlockSpec((F, D), lambda i: (0, 0)),
            pl.BlockSpec((F, D), lambda i: (0, 0)),
            pl.BlockSpec((D, F), lambda i: (0, 0)),
            pl.BlockSpec((ROWS, TOP_K, D), lambda i: (i, 0, 0)),
        ],
        out_specs=pl.BlockSpec((ROWS, D), lambda i: (i, 0)),
        out_shape=jax.ShapeDtypeStruct((N, D), jnp.float32),
        compiler_params=pltpu.CompilerParams(
            dimension_semantics=("arbitrary",)),
    )(x_flat, s_fc1_w, s_fc2_w, s_proj_w, routed_pairs)

    return y.reshape(B, T, D)


# speedup vs baseline: 1.2598x; 1.0717x over previous
import jax
import jax.numpy as jnp
from jax.experimental import pallas as pl
from jax.experimental.pallas import tpu as pltpu

N_EXPERT = 8
TOP_K = 2
ROWS = 256


def _router_body(x_ref, gw_ref, o_ref):
    o_ref[...] = jax.lax.dot_general(
        x_ref[...], gw_ref[...], (((1,), (1,)), ((), ())),
        preferred_element_type=jnp.float32)


def _fc12_body(be_ref, x_ref, w1_ref, w2_ref, h_ref):
    x = x_ref[...]
    h1 = jax.lax.dot_general(x, w1_ref[0], (((1,), (1,)), ((), ())),
                             preferred_element_type=jnp.float32)
    h2 = jax.lax.dot_general(x, w2_ref[0], (((1,), (1,)), ((), ())),
                             preferred_element_type=jnp.float32)
    h_ref[...] = jax.nn.silu(h1) * h2


def _proj_body(be_ref, h_ref, wp_ref, wt_ref, o_ref):
    out = jax.lax.dot_general(h_ref[...], wp_ref[0], (((1,), (1,)), ((), ())),
                              preferred_element_type=jnp.float32)
    o_ref[...] = out * wt_ref[0, 0].reshape(-1, 1)


def _shared_body(x_ref, w1_ref, w2_ref, wp_ref, r_ref, o_ref):
    x = x_ref[...]
    h1 = jax.lax.dot_general(x, w1_ref[...], (((1,), (1,)), ((), ())),
                             preferred_element_type=jnp.float32)
    h2 = jax.lax.dot_general(x, w2_ref[...], (((1,), (1,)), ((), ())),
                             preferred_element_type=jnp.float32)
    h = jax.nn.silu(h1) * h2
    out = jax.lax.dot_general(h, wp_ref[...], (((1,), (1,)), ((), ())),
                              preferred_element_type=jnp.float32)
    routed = r_ref[...]
    o_ref[...] = out + routed[:, 0, :] + routed[:, 1, :]


def kernel(x, gate_w, fc1_w, fc2_w, proj_w, s_fc1_w, s_fc2_w, s_proj_w):
    B, T, D = x.shape
    N = B * T
    F = fc1_w.shape[1]
    A = N * TOP_K
    PAD = A + N_EXPERT * ROWS
    n_blocks = PAD // ROWS

    x_flat = x.reshape(N, D)

    logits = pl.pallas_call(
        _router_body,
        grid=(N // 512,),
        in_specs=[pl.BlockSpec((512, D), lambda i: (i, 0)),
                  pl.BlockSpec((N_EXPERT, D), lambda i: (0, 0))],
        out_specs=pl.BlockSpec((512, N_EXPERT), lambda i: (i, 0)),
        out_shape=jax.ShapeDtypeStruct((N, N_EXPERT), jnp.float32),
    )(x_flat, gate_w)

    top_vals, top_idx = jax.lax.top_k(logits, TOP_K)
    probs = jax.nn.softmax(top_vals, axis=-1)

    ew = top_idx.reshape(-1).astype(jnp.int32)
    wts = probs.reshape(-1)
    tok = jax.lax.iota(jnp.int32, A) // TOP_K

    oh = (ew[:, None] == jax.lax.iota(jnp.int32, N_EXPERT)[None, :]
          ).astype(jnp.int32)
    counts = oh.sum(axis=0)
    rank = (jnp.cumsum(oh, axis=0) * oh).sum(axis=1) - 1
    padded = ((counts + ROWS - 1) // ROWS) * ROWS
    pstart = jnp.concatenate([jnp.zeros((1,), jnp.int32),
                              jnp.cumsum(padded).astype(jnp.int32)])
    dest = pstart[ew] + rank

    tok_pad = jnp.zeros((PAD,), jnp.int32).at[dest].set(tok)
    wt_pad = jnp.zeros((PAD,), jnp.float32).at[dest].set(wts)
    inv = dest

    blk_ids = jax.lax.iota(jnp.int32, n_blocks) * ROWS
    block_expert = jnp.minimum(
        jnp.searchsorted(pstart[1:], blk_ids, side="right"),
        N_EXPERT - 1).astype(jnp.int32)

    x_g = x_flat[tok_pad]

    h_pad = pl.pallas_call(
        _fc12_body,
        grid_spec=pltpu.PrefetchScalarGridSpec(
            num_scalar_prefetch=1,
            grid=(n_blocks,),
            in_specs=[
                pl.BlockSpec((ROWS, D), lambda i, be: (i, 0)),
                pl.BlockSpec((1, F, D), lambda i, be: (be[i], 0, 0)),
                pl.BlockSpec((1, F, D), lambda i, be: (be[i], 0, 0)),
            ],
            out_specs=pl.BlockSpec((ROWS, F), lambda i, be: (i, 0)),
        ),
        out_shape=jax.ShapeDtypeStruct((PAD, F), jnp.float32),
        compiler_params=pltpu.CompilerParams(
            dimension_semantics=("arbitrary",)),
    )(block_expert, x_g, fc1_w, fc2_w)

    out_pad = pl.pallas_call(
        _proj_body,
        grid_spec=pltpu.PrefetchScalarGridSpec(
            num_scalar_prefetch=1,
            grid=(n_blocks,),
            in_specs=[
                pl.BlockSpec((ROWS, F), lambda i, be: (i, 0)),
                pl.BlockSpec((1, D, F), lambda i, be: (be[i], 0, 0)),
                pl.BlockSpec((1, 1, ROWS), lambda i, be: (i, 0, 0)),
            ],
            out_specs=pl.BlockSpec((ROWS, D), lambda i, be: (i, 0)),
        ),
        out_shape=jax.ShapeDtypeStruct((PAD, D), jnp.float32),
        compiler_params=pltpu.CompilerParams(
            dimension_semantics=("arbitrary",)),
    )(block_expert, h_pad, proj_w, wt_pad.reshape(n_blocks, 1, ROWS))

    routed_pairs = out_pad[inv].reshape(N, TOP_K, D)

    n_sblocks = N // ROWS
    y = pl.pallas_call(
        _shared_body,
        grid=(n_sblocks,),
        in_specs=[
            pl.BlockSpec((ROWS, D), lambda i: (i, 0)),
            pl.BlockSpec((F, D), lambda i: (0, 0)),
            pl.BlockSpec((F, D), lambda i: (0, 0)),
            pl.BlockSpec((D, F), lambda i: (0, 0)),
            pl.BlockSpec((ROWS, TOP_K, D), lambda i: (i, 0, 0)),
        ],
        out_specs=pl.BlockSpec((ROWS, D), lambda i: (i, 0)),
        out_shape=jax.ShapeDtypeStruct((N, D), jnp.float32),
        compiler_params=pltpu.CompilerParams(
            dimension_semantics=("arbitrary",)),
    )(x_flat, s_fc1_w, s_fc2_w, s_proj_w, routed_pairs)

    return y.reshape(B, T, D)
